# Initial kernel scaffold; baseline (speedup 1.0000x reference)
import jax
import jax.numpy as jnp
from jax.experimental import pallas as pl


def kernel(x, edge_index, batch, W1, b1, W2, b2, W3, b3, W4, b4, Wres, bres, ln_g, ln_b, fcn_g, fcn_b, fc1_W, fc1_b, fc2_W, fc2_b):
    raise NotImplementedError("write your pallas kernel here")


# speedup vs baseline: 12.9535x; 12.9535x over previous
import functools

import jax
import jax.numpy as jnp
from jax import lax
from jax.experimental import pallas as pl
from jax.experimental.pallas import tpu as pltpu
from jax.experimental.pallas import tpu_sc as plsc

_N = 10000
_NPAD = 10240
_EPAD = 327680
_NC = 2
_NS = 16
_K = 128
_BN = 1000
_GRAPHS = 64

_f32 = jnp.float32
_SDS = jax.ShapeDtypeStruct
_MESH = dict(core_axis_name="c", subcore_axis_name="s")


def _leaky(v):
    return jnp.where(v >= 0, v, 0.01 * v)


def _make_deg_kernel(npad=_NPAD, epad=_EPAD):
    ept = epad // (_NC * _NS)
    iters = ept // _K
    rpt = npad // _NS
    mesh = plsc.VectorSubcoreMesh(**_MESH)

    @functools.partial(
        pl.kernel, mesh=mesh,
        out_type=_SDS((_NC, npad), _f32),
        scratch_types=[
            pltpu.VMEM((_K,), jnp.int32),
            pltpu.VMEM((_K,), _f32),
            pltpu.VMEM((rpt,), _f32),
            pltpu.VMEM_SHARED((npad,), _f32),
            pltpu.SemaphoreType.DMA,
        ],
    )
    def deg_kernel(col_h, zeros_h, out_h, cidx, ones_v, tmp_v, acc, sem):
        c = lax.axis_index("c")
        s = lax.axis_index("s")
        wid = s * _NC + c
        for j in range(_K // 16):
            ones_v[pl.ds(j * 16, 16)] = jnp.full((16,), 1.0, _f32)
        pltpu.sync_copy(zeros_h, tmp_v)
        pltpu.sync_copy(tmp_v, acc.at[pl.ds(s * rpt, rpt)])
        plsc.subcore_barrier()

        def body(i, carry):
            off = wid * ept + i * _K
            pltpu.sync_copy(col_h.at[pl.ds(off, _K)], cidx)
            pltpu.sync_copy(ones_v, acc.at[cidx], add=True)
            return carry

        lax.fori_loop(0, iters, body, 0)
        plsc.subcore_barrier()
        pltpu.sync_copy(acc.at[pl.ds(s * rpt, rpt)], tmp_v)
        pltpu.sync_copy(tmp_v, out_h.at[c, pl.ds(s * rpt, rpt)])

    return deg_kernel


def _make_agg_e(npad=_NPAD, epad=_EPAD, fc=128):
    ept = epad // (_NC * _NS)
    iters = ept // _K
    rpt = npad // _NS
    rk = min(_K, rpt)
    mesh = plsc.VectorSubcoreMesh(**_MESH)

    @functools.partial(
        pl.kernel, mesh=mesh,
        out_type=_SDS((_NC, npad, fc), _f32),
        scratch_types=[
            pltpu.VMEM((_K,), jnp.int32),
            pltpu.VMEM((_K,), jnp.int32),
            pltpu.VMEM((_K, fc), _f32),
            pltpu.VMEM_SHARED((npad, fc), _f32),
            pltpu.SemaphoreType.DMA,
        ],
    )
    def agg_kernel(gs, row_h, col_h, zeros_h, out_h,
                   ridx, cidx, rows, acc, sem):
        c = lax.axis_index("c")
        s = lax.axis_index("s")
        wid = s * _NC + c
        pltpu.sync_copy(zeros_h.at[pl.ds(0, rk)], rows.at[pl.ds(0, rk)])
        for j in range(rpt // rk):
            pltpu.sync_copy(rows.at[pl.ds(0, rk)],
                            acc.at[pl.ds(s * rpt + j * rk, rk)])
        plsc.subcore_barrier()

        def body(i, carry):
            off = wid * ept + i * _K
            pltpu.sync_copy(row_h.at[pl.ds(off, _K)], ridx)
            pltpu.sync_copy(col_h.at[pl.ds(off, _K)], cidx)
            pltpu.async_copy(gs.at[ridx], rows, sem).wait()
            pltpu.sync_copy(rows, acc.at[cidx], add=True)
            return carry

        lax.fori_loop(0, iters, body, 0)
        plsc.subcore_barrier()
        for j in range(rpt // rk):
            base = s * rpt + j * rk
            pltpu.sync_copy(acc.at[pl.ds(base, rk)], rows.at[pl.ds(0, rk)])
            pltpu.sync_copy(rows.at[pl.ds(0, rk)],
                            out_h.at[c, pl.ds(base, rk)])

    return agg_kernel


def _make_agg_f(npad=_NPAD, epad=_EPAD, fc=128):
    ept = epad // _NS
    iters = ept // _K
    rpt = npad // _NS
    rk = min(_K, rpt)
    mesh = plsc.VectorSubcoreMesh(**_MESH)

    @functools.partial(
        pl.kernel, mesh=mesh,
        out_type=_SDS((_NC, npad, fc), _f32),
        scratch_types=[
            pltpu.VMEM((_K,), jnp.int32),
            pltpu.VMEM((_K,), jnp.int32),
            pltpu.VMEM((_K, fc), _f32),
            pltpu.VMEM_SHARED((npad, fc), _f32),
            pltpu.SemaphoreType.DMA,
        ],
    )
    def agg_kernel(gs2n, row2_h, col_h, zeros_h, out_h,
                   ridx, cidx, rows, acc, sem):
        c = lax.axis_index("c")
        s = lax.axis_index("s")
        pltpu.sync_copy(zeros_h.at[pl.ds(0, rk)], rows.at[pl.ds(0, rk)])
        for j in range(rpt // rk):
            pltpu.sync_copy(rows.at[pl.ds(0, rk)],
                            acc.at[pl.ds(s * rpt + j * rk, rk)])
        plsc.subcore_barrier()

        def body(i, carry):
            off = s * ept + i * _K
            pltpu.sync_copy(row2_h.at[c, pl.ds(off, _K)], ridx)
            pltpu.sync_copy(col_h.at[pl.ds(off, _K)], cidx)
            pltpu.async_copy(gs2n.at[ridx], rows, sem).wait()
            pltpu.sync_copy(rows, acc.at[cidx], add=True)
            return carry

        lax.fori_loop(0, iters, body, 0)
        plsc.subcore_barrier()
        for j in range(rpt // rk):
            base = s * rpt + j * rk
            pltpu.sync_copy(acc.at[pl.ds(base, rk)], rows.at[pl.ds(0, rk)])
            pltpu.sync_copy(rows.at[pl.ds(0, rk)],
                            out_h.at[c, pl.ds(base, rk)])

    return agg_kernel


def _row_spec(w):
    return pl.BlockSpec((_BN, w), lambda i: (i, 0))


def _full_spec(h, w):
    return pl.BlockSpec((h, w), lambda i: (0, 0))


def _plane_spec(p, w):
    return pl.BlockSpec((1, _BN, w), lambda i, _p=p: (_p, i, 0))


def _mm1_body(x, wcat, bres, dcol, gs1, res):
    h = jnp.dot(x[...], wcat[...], preferred_element_type=_f32)
    dinv = lax.rsqrt(dcol[...] + 1.0)
    gs1[...] = jnp.concatenate(
        [dinv * h[:, :64], jnp.zeros((_BN, 64), _f32)], axis=1)
    res[...] = h[:, 64:] + bres[...]


def _ew2_body(gs1, a0, a1, b1, dcol, gs2):
    dinv = lax.rsqrt(dcol[...] + 1.0)
    acc = a0[...][0] + a1[...][0]
    h1 = _leaky(dinv * (acc[:, :64] + gs1[:, :64]) + b1[...])
    gs2[...] = jnp.concatenate(
        [dinv * h1, jnp.zeros((_BN, 64), _f32)], axis=1)


def _mm2_body(gs2, a0, a1, w2, b2, dcol, gs3):
    dinv = lax.rsqrt(dcol[...] + 1.0)
    acc = a0[...][0] + a1[...][0]
    a2 = dinv * (acc[:, :64] + gs2[:, :64])
    h2 = _leaky(jnp.dot(a2, w2[...], preferred_element_type=_f32) + b2[...])
    g = dinv * h2
    gs3[...] = jnp.stack([g[:, :128], g[:, 128:]], axis=0)


def _mm34_body(gs3, f0, f1, w3, b3, w4, dcol, gs4):
    dinv = lax.rsqrt(dcol[...] + 1.0)
    g3 = gs3[...]
    a3 = dinv * jnp.concatenate(
        [f0[...][0] + g3[0], f1[...][0] + g3[1]], axis=1)
    h3 = _leaky(jnp.dot(a3, w3[...], preferred_element_type=_f32) + b3[...])
    m4 = jnp.dot(h3, w4[...], preferred_element_type=_f32)
    gs4[...] = jnp.concatenate(
        [dinv * m4, jnp.zeros((_BN, 64), _f32)], axis=1)


def _fin_body(gs4, a0, a1, b4, resi, lng, lnb, dcol, z):
    dinv = lax.rsqrt(dcol[...] + 1.0)
    acc = a0[...][0] + a1[...][0]
    h4 = _leaky(dinv * (acc[:, :64] + gs4[:, :64]) + b4[...])
    y = h4 + resi[...]
    m = jnp.mean(y, axis=-1, keepdims=True)
    v = jnp.mean((y - m) * (y - m), axis=-1, keepdims=True)
    z[...] = (y - m) * lax.rsqrt(v + 1e-5) * lng[...] + lnb[...]


def _pool_head_body(zb, bb, fc1w, fc1b, fc2w, fc2b, fcng, fcnb, out,
                    pool_acc):
    i = pl.program_id(0)
    nsteps = pl.num_programs(0)

    @pl.when(i == 0)
    def _():
        pool_acc[...] = jnp.full((_GRAPHS, 64), -jnp.inf, _f32)

    zv = zb[...]
    bv = bb[...]
    cur = pool_acc[...]
    rows = []
    for g in range(_GRAPHS):
        mask = bv == g
        rows.append(jnp.max(jnp.where(mask, zv, -jnp.inf), axis=0,
                            keepdims=True))
    pool_acc[...] = jnp.maximum(cur, jnp.concatenate(rows, axis=0))

    @pl.when(i == nsteps - 1)
    def _():
        p = pool_acc[...]
        h = jnp.dot(p, fc1w[...], preferred_element_type=_f32) + fc1b[...]
        m = jnp.mean(h, axis=-1, keepdims=True)
        v = jnp.mean((h - m) * (h - m), axis=-1, keepdims=True)
        h = (h - m) * lax.rsqrt(v + 1e-5) * fcng[...] + fcnb[...]
        h = _leaky(h)
        out[...] = jnp.dot(h, fc2w[...], preferred_element_type=_f32) \
            + fc2b[...]


def kernel(x, edge_index, batch, W1, b1, W2, b2, W3, b3, W4, b4, Wres, bres,
           ln_g, ln_b, fcn_g, fcn_b, fc1_W, fc1_b, fc2_W, fc2_b):
    n = x.shape[0]
    e = edge_index.shape[1]
    grid = (n // _BN,)

    pad = _EPAD - e
    ar = jnp.arange(pad, dtype=jnp.int32)
    row_pad = jnp.concatenate([edge_index[0], ar % n])
    col_pad = jnp.concatenate([edge_index[1], n + (ar % (_NPAD - n))])
    row2 = jnp.stack([row_pad, row_pad + jnp.int32(n)])
    zeros1 = jnp.zeros((_NPAD // _NS,), _f32)
    zeros128 = jnp.zeros((_K, 128), _f32)
    wcat = jnp.concatenate([W1, Wres], axis=1)
    b1r = b1.reshape(1, 64)
    b2r = b2.reshape(1, 256)
    b3r = b3.reshape(1, 512)
    b4r = b4.reshape(1, 64)
    bresr = bres.reshape(1, 64)
    lngr = ln_g.reshape(1, 64)
    lnbr = ln_b.reshape(1, 64)
    fcngr = fcn_g.reshape(1, 64)
    fcnbr = fcn_b.reshape(1, 64)
    fc1br = fc1_b.reshape(1, 64)
    fc2br = fc2_b.reshape(1, 16)
    batch2 = batch.reshape(n, 1)

    degp = _make_deg_kernel()(col_pad, zeros1)
    dcol = (degp[0, :n] + degp[1, :n]).reshape(n, 1)

    dspec = _row_spec(1)
    agg64 = _make_agg_e()
    p0s = _plane_spec(0, 128)
    p1s = _plane_spec(1, 128)

    gs1, res = pl.pallas_call(
        _mm1_body,
        grid=grid,
        in_specs=[_row_spec(128), _full_spec(128, 128), _full_spec(1, 64),
                  dspec],
        out_specs=[_row_spec(128), _row_spec(64)],
        out_shape=[_SDS((n, 128), _f32), _SDS((n, 64), _f32)],
    )(x, wcat, bresr, dcol)

    acc1 = agg64(gs1, row_pad, col_pad, zeros128)

    gs2 = pl.pallas_call(
        _ew2_body,
        grid=grid,
        in_specs=[_row_spec(128), p0s, p1s, _full_spec(1, 64), dspec],
        out_specs=_row_spec(128),
        out_shape=_SDS((n, 128), _f32),
    )(gs1, acc1, acc1, b1r, dcol)

    acc2 = agg64(gs2, row_pad, col_pad, zeros128)

    gs3 = pl.pallas_call(
        _mm2_body,
        grid=grid,
        in_specs=[_row_spec(128), p0s, p1s, _full_spec(64, 256),
                  _full_spec(1, 256), dspec],
        out_specs=pl.BlockSpec((2, _BN, 128), lambda i: (0, i, 0)),
        out_shape=_SDS((2, n, 128), _f32),
    )(gs2, acc2, acc2, W2, b2r, dcol)

    gs3cat = gs3.reshape(2 * n, 128)
    acc3 = _make_agg_f()(gs3cat, row2, col_pad, zeros128)

    gs4 = pl.pallas_call(
        _mm34_body,
        grid=grid,
        in_specs=[pl.BlockSpec((2, _BN, 128), lambda i: (0, i, 0)),
                  p0s, p1s, _full_spec(256, 512), _full_spec(1, 512),
                  _full_spec(512, 64), dspec],
        out_specs=_row_spec(128),
        out_shape=_SDS((n, 128), _f32),
    )(gs3, acc3, acc3, W3, b3r, W4, dcol)

    acc4 = agg64(gs4, row_pad, col_pad, zeros128)

    z = pl.pallas_call(
        _fin_body,
        grid=grid,
        in_specs=[_row_spec(128), p0s, p1s, _full_spec(1, 64), _row_spec(64),
                  _full_spec(1, 64), _full_spec(1, 64), dspec],
        out_specs=_row_spec(64),
        out_shape=_SDS((n, 64), _f32),
    )(gs4, acc4, acc4, b4r, res, lngr, lnbr, dcol)

    out = pl.pallas_call(
        _pool_head_body,
        grid=grid,
        in_specs=[_row_spec(64), pl.BlockSpec((_BN, 1), lambda i: (i, 0)),
                  _full_spec(64, 64), _full_spec(1, 64), _full_spec(64, 16),
                  _full_spec(1, 16), _full_spec(1, 64), _full_spec(1, 64)],
        out_specs=_full_spec(_GRAPHS, 16),
        out_shape=_SDS((_GRAPHS, 16), _f32),
        scratch_shapes=[pltpu.VMEM((_GRAPHS, 64), _f32)],
        compiler_params=pltpu.CompilerParams(
            dimension_semantics=("arbitrary",)),
    )(z, batch2, fc1_W, fc1br, fc2_W, fc2br, fcngr, fcnbr)

    return out


# speedup vs baseline: 21.4083x; 1.6527x over previous
import functools

import jax
import jax.numpy as jnp
from jax import lax
from jax.experimental import pallas as pl
from jax.experimental.pallas import tpu as pltpu
from jax.experimental.pallas import tpu_sc as plsc

_N = 10000
_NPAD = 10240
_EPAD = 327680
_NC = 2
_NS = 16
_K = 128
_BN = 1000
_GRAPHS = 64

_f32 = jnp.float32
_SDS = jax.ShapeDtypeStruct
_MESH = dict(core_axis_name="c", subcore_axis_name="s")


def _leaky(v):
    return jnp.where(v >= 0, v, 0.01 * v)


def _make_deg_kernel(npad=_NPAD, epad=_EPAD):
    ept = epad // (_NC * _NS)
    iters = ept // _K
    rpt = npad // _NS
    mesh = plsc.VectorSubcoreMesh(**_MESH)

    @functools.partial(
        pl.kernel, mesh=mesh,
        out_type=_SDS((_NC, npad), _f32),
        scratch_types=[
            pltpu.VMEM((_K,), jnp.int32),
            pltpu.VMEM((_K,), _f32),
            pltpu.VMEM((rpt,), _f32),
            pltpu.VMEM_SHARED((npad,), _f32),
            pltpu.SemaphoreType.DMA,
        ],
    )
    def deg_kernel(col_h, zeros_h, out_h, cidx, ones_v, tmp_v, acc, sem):
        c = lax.axis_index("c")
        s = lax.axis_index("s")
        wid = s * _NC + c
        for j in range(_K // 16):
            ones_v[pl.ds(j * 16, 16)] = jnp.full((16,), 1.0, _f32)
        pltpu.sync_copy(zeros_h, tmp_v)
        pltpu.sync_copy(tmp_v, acc.at[pl.ds(s * rpt, rpt)])
        plsc.subcore_barrier()

        def body(i, carry):
            off = wid * ept + i * _K
            pltpu.sync_copy(col_h.at[pl.ds(off, _K)], cidx)
            pltpu.sync_copy(ones_v, acc.at[cidx], add=True)
            return carry

        lax.fori_loop(0, iters, body, 0)
        plsc.subcore_barrier()
        pltpu.sync_copy(acc.at[pl.ds(s * rpt, rpt)], tmp_v)
        pltpu.sync_copy(tmp_v, out_h.at[c, pl.ds(s * rpt, rpt)])

    return deg_kernel


_NB = 2
_U = 8
_f32 = jnp.float32
_SDS = jax.ShapeDtypeStruct
_MESH = dict(core_axis_name="c", subcore_axis_name="s")


def _pipelined_loop(gs, row_src, col_src, acc, ridx8, cidx8, bufs, gsems,
                    ssems, cbase, iters8):
    def outer(i8, carry):
        off = cbase + i8 * _U
        pltpu.sync_copy(row_src.at[pl.ds(off, _U)], ridx8)
        pltpu.sync_copy(col_src.at[pl.ds(off, _U)], cidx8)
        hg = {}
        hs = {}
        for j in range(_NB):
            hg[j] = pltpu.async_copy(gs.at[ridx8.at[j]], bufs[j], gsems[j])
        for j in range(_U):
            b = j % _NB
            hg[j].wait()
            hs[j] = pltpu.async_copy(bufs[b], acc.at[cidx8.at[j]], ssems[b],
                                     add=True)
            nj = j + _NB
            if nj < _U:
                hs[j].wait()
                hg[nj] = pltpu.async_copy(gs.at[ridx8.at[nj]], bufs[b],
                                          gsems[b])
        for j in range(_U - _NB, _U):
            hs[j].wait()
        return carry

    lax.fori_loop(0, iters8, outer, 0)


def _make_agg_e(npad=_NPAD, epad=_EPAD, fc=128):
    ept = epad // (_NC * _NS)
    iters8 = ept // (_U * _K)
    rpt = npad // _NS
    rk = min(_K, rpt)
    mesh = plsc.VectorSubcoreMesh(**_MESH)

    @functools.partial(
        pl.kernel, mesh=mesh,
        out_type=_SDS((_NC, npad, fc), _f32),
        scratch_types=[
            pltpu.VMEM((_U, _K), jnp.int32),
            pltpu.VMEM((_U, _K), jnp.int32),
        ] + [pltpu.VMEM((_K, fc), _f32) for _ in range(_NB)] + [
            pltpu.VMEM_SHARED((npad, fc), _f32),
        ] + [pltpu.SemaphoreType.DMA for _ in range(2 * _NB)],
    )
    def agg_kernel(gs, row_h, col_h, zeros_h, out_h,
                   ridx8, cidx8, b0, b1, acc, g0, g1, s0, s1):
        c = lax.axis_index("c")
        s = lax.axis_index("s")
        wid = s * _NC + c
        bufs = [b0, b1]
        gsems = [g0, g1]
        ssems = [s0, s1]
        rows = b0
        pltpu.sync_copy(zeros_h.at[pl.ds(0, rk)], rows.at[pl.ds(0, rk)])
        for j in range(rpt // rk):
            pltpu.sync_copy(rows.at[pl.ds(0, rk)],
                            acc.at[pl.ds(s * rpt + j * rk, rk)])
        plsc.subcore_barrier()
        _pipelined_loop(gs, row_h, col_h, acc, ridx8, cidx8, bufs, gsems,
                        ssems, wid * (ept // _K), iters8)
        plsc.subcore_barrier()
        for j in range(rpt // rk):
            base = s * rpt + j * rk
            pltpu.sync_copy(acc.at[pl.ds(base, rk)], rows.at[pl.ds(0, rk)])
            pltpu.sync_copy(rows.at[pl.ds(0, rk)],
                            out_h.at[c, pl.ds(base, rk)])

    return agg_kernel


def _make_agg_f(npad=_NPAD, epad=_EPAD, fc=128):
    ept = epad // _NS
    iters8 = ept // (_U * _K)
    rpt = npad // _NS
    rk = min(_K, rpt)
    mesh = plsc.VectorSubcoreMesh(**_MESH)

    @functools.partial(
        pl.kernel, mesh=mesh,
        out_type=_SDS((_NC, npad, fc), _f32),
        scratch_types=[
            pltpu.VMEM((_U, _K), jnp.int32),
            pltpu.VMEM((_U, _K), jnp.int32),
        ] + [pltpu.VMEM((_K, fc), _f32) for _ in range(_NB)] + [
            pltpu.VMEM_SHARED((npad, fc), _f32),
        ] + [pltpu.SemaphoreType.DMA for _ in range(2 * _NB)],
    )
    def agg_kernel(gs2n, row2_h, col_h, zeros_h, out_h,
                   ridx8, cidx8, b0, b1, acc, g0, g1, s0, s1):
        c = lax.axis_index("c")
        s = lax.axis_index("s")
        bufs = [b0, b1]
        gsems = [g0, g1]
        ssems = [s0, s1]
        rows = b0
        pltpu.sync_copy(zeros_h.at[pl.ds(0, rk)], rows.at[pl.ds(0, rk)])
        for j in range(rpt // rk):
            pltpu.sync_copy(rows.at[pl.ds(0, rk)],
                            acc.at[pl.ds(s * rpt + j * rk, rk)])
        plsc.subcore_barrier()
        _pipelined_loop(gs2n, row2_h.at[c], col_h, acc, ridx8, cidx8, bufs,
                        gsems, ssems, s * (ept // _K), iters8)
        plsc.subcore_barrier()
        for j in range(rpt // rk):
            base = s * rpt + j * rk
            pltpu.sync_copy(acc.at[pl.ds(base, rk)], rows.at[pl.ds(0, rk)])
            pltpu.sync_copy(rows.at[pl.ds(0, rk)],
                            out_h.at[c, pl.ds(base, rk)])

    return agg_kernel


def _row_spec(w):
    return pl.BlockSpec((_BN, w), lambda i: (i, 0))


def _full_spec(h, w):
    return pl.BlockSpec((h, w), lambda i: (0, 0))


def _plane_spec(p, w):
    return pl.BlockSpec((1, _BN, w), lambda i, _p=p: (_p, i, 0))


def _mm1_body(x, wcat, bres, dcol, gs1, res):
    h = jnp.dot(x[...], wcat[...], preferred_element_type=_f32)
    dinv = lax.rsqrt(dcol[...] + 1.0)
    gs1[...] = jnp.concatenate(
        [dinv * h[:, :64], jnp.zeros((_BN, 64), _f32)], axis=1)
    res[...] = h[:, 64:] + bres[...]


def _ew2_body(gs1, a0, a1, b1, dcol, gs2):
    dinv = lax.rsqrt(dcol[...] + 1.0)
    acc = a0[...][0] + a1[...][0]
    h1 = _leaky(dinv * (acc[:, :64] + gs1[:, :64]) + b1[...])
    gs2[...] = jnp.concatenate(
        [dinv * h1, jnp.zeros((_BN, 64), _f32)], axis=1)


def _mm2_body(gs2, a0, a1, w2, b2, dcol, gs3):
    dinv = lax.rsqrt(dcol[...] + 1.0)
    acc = a0[...][0] + a1[...][0]
    a2 = dinv * (acc[:, :64] + gs2[:, :64])
    h2 = _leaky(jnp.dot(a2, w2[...], preferred_element_type=_f32) + b2[...])
    g = dinv * h2
    gs3[...] = jnp.stack([g[:, :128], g[:, 128:]], axis=0)


def _mm34_body(gs3, f0, f1, w3, b3, w4, dcol, gs4):
    dinv = lax.rsqrt(dcol[...] + 1.0)
    g3 = gs3[...]
    a3 = dinv * jnp.concatenate(
        [f0[...][0] + g3[0], f1[...][0] + g3[1]], axis=1)
    h3 = _leaky(jnp.dot(a3, w3[...], preferred_element_type=_f32) + b3[...])
    m4 = jnp.dot(h3, w4[...], preferred_element_type=_f32)
    gs4[...] = jnp.concatenate(
        [dinv * m4, jnp.zeros((_BN, 64), _f32)], axis=1)


def _fin_body(gs4, a0, a1, b4, resi, lng, lnb, dcol, z):
    dinv = lax.rsqrt(dcol[...] + 1.0)
    acc = a0[...][0] + a1[...][0]
    h4 = _leaky(dinv * (acc[:, :64] + gs4[:, :64]) + b4[...])
    y = h4 + resi[...]
    m = jnp.mean(y, axis=-1, keepdims=True)
    v = jnp.mean((y - m) * (y - m), axis=-1, keepdims=True)
    z[...] = (y - m) * lax.rsqrt(v + 1e-5) * lng[...] + lnb[...]


def _pool_head_body(zb, bb, fc1w, fc1b, fc2w, fc2b, fcng, fcnb, out,
                    pool_acc):
    i = pl.program_id(0)
    nsteps = pl.num_programs(0)

    @pl.when(i == 0)
    def _():
        pool_acc[...] = jnp.full((_GRAPHS, 64), -jnp.inf, _f32)

    zv = zb[...]
    bv = bb[...]
    cur = pool_acc[...]
    rows = []
    for g in range(_GRAPHS):
        mask = bv == g
        rows.append(jnp.max(jnp.where(mask, zv, -jnp.inf), axis=0,
                            keepdims=True))
    pool_acc[...] = jnp.maximum(cur, jnp.concatenate(rows, axis=0))

    @pl.when(i == nsteps - 1)
    def _():
        p = pool_acc[...]
        h = jnp.dot(p, fc1w[...], preferred_element_type=_f32) + fc1b[...]
        m = jnp.mean(h, axis=-1, keepdims=True)
        v = jnp.mean((h - m) * (h - m), axis=-1, keepdims=True)
        h = (h - m) * lax.rsqrt(v + 1e-5) * fcng[...] + fcnb[...]
        h = _leaky(h)
        out[...] = jnp.dot(h, fc2w[...], preferred_element_type=_f32) \
            + fc2b[...]


def kernel(x, edge_index, batch, W1, b1, W2, b2, W3, b3, W4, b4, Wres, bres,
           ln_g, ln_b, fcn_g, fcn_b, fc1_W, fc1_b, fc2_W, fc2_b):
    n = x.shape[0]
    e = edge_index.shape[1]
    grid = (n // _BN,)

    pad = _EPAD - e
    ar = jnp.arange(pad, dtype=jnp.int32)
    row_pad = jnp.concatenate([edge_index[0], ar % n])
    col_pad = jnp.concatenate([edge_index[1], n + (ar % (_NPAD - n))])
    row2 = jnp.stack([row_pad, row_pad + jnp.int32(n)])
    zeros1 = jnp.zeros((_NPAD // _NS,), _f32)
    zeros128 = jnp.zeros((_K, 128), _f32)
    wcat = jnp.concatenate([W1, Wres], axis=1)
    b1r = b1.reshape(1, 64)
    b2r = b2.reshape(1, 256)
    b3r = b3.reshape(1, 512)
    b4r = b4.reshape(1, 64)
    bresr = bres.reshape(1, 64)
    lngr = ln_g.reshape(1, 64)
    lnbr = ln_b.reshape(1, 64)
    fcngr = fcn_g.reshape(1, 64)
    fcnbr = fcn_b.reshape(1, 64)
    fc1br = fc1_b.reshape(1, 64)
    fc2br = fc2_b.reshape(1, 16)
    batch2 = batch.reshape(n, 1)

    degp = _make_deg_kernel()(col_pad, zeros1)
    dcol = (degp[0, :n] + degp[1, :n]).reshape(n, 1)

    dspec = _row_spec(1)
    agg64 = _make_agg_e()
    p0s = _plane_spec(0, 128)
    p1s = _plane_spec(1, 128)

    gs1, res = pl.pallas_call(
        _mm1_body,
        grid=grid,
        in_specs=[_row_spec(128), _full_spec(128, 128), _full_spec(1, 64),
                  dspec],
        out_specs=[_row_spec(128), _row_spec(64)],
        out_shape=[_SDS((n, 128), _f32), _SDS((n, 64), _f32)],
    )(x, wcat, bresr, dcol)

    row2d = row_pad.reshape(_EPAD // _K, _K)
    col2d = col_pad.reshape(_EPAD // _K, _K)
    row23d = row2.reshape(2, _EPAD // _K, _K)

    acc1 = agg64(gs1, row2d, col2d, zeros128)

    gs2 = pl.pallas_call(
        _ew2_body,
        grid=grid,
        in_specs=[_row_spec(128), p0s, p1s, _full_spec(1, 64), dspec],
        out_specs=_row_spec(128),
        out_shape=_SDS((n, 128), _f32),
    )(gs1, acc1, acc1, b1r, dcol)

    acc2 = agg64(gs2, row2d, col2d, zeros128)

    gs3 = pl.pallas_call(
        _mm2_body,
        grid=grid,
        in_specs=[_row_spec(128), p0s, p1s, _full_spec(64, 256),
                  _full_spec(1, 256), dspec],
        out_specs=pl.BlockSpec((2, _BN, 128), lambda i: (0, i, 0)),
        out_shape=_SDS((2, n, 128), _f32),
    )(gs2, acc2, acc2, W2, b2r, dcol)

    gs3cat = gs3.reshape(2 * n, 128)
    acc3 = _make_agg_f()(gs3cat, row23d, col2d, zeros128)

    gs4 = pl.pallas_call(
        _mm34_body,
        grid=grid,
        in_specs=[pl.BlockSpec((2, _BN, 128), lambda i: (0, i, 0)),
                  p0s, p1s, _full_spec(256, 512), _full_spec(1, 512),
                  _full_spec(512, 64), dspec],
        out_specs=_row_spec(128),
        out_shape=_SDS((n, 128), _f32),
    )(gs3, acc3, acc3, W3, b3r, W4, dcol)

    acc4 = agg64(gs4, row2d, col2d, zeros128)

    z = pl.pallas_call(
        _fin_body,
        grid=grid,
        in_specs=[_row_spec(128), p0s, p1s, _full_spec(1, 64), _row_spec(64),
                  _full_spec(1, 64), _full_spec(1, 64), dspec],
        out_specs=_row_spec(64),
        out_shape=_SDS((n, 64), _f32),
    )(gs4, acc4, acc4, b4r, res, lngr, lnbr, dcol)

    out = pl.pallas_call(
        _pool_head_body,
        grid=grid,
        in_specs=[_row_spec(64), pl.BlockSpec((_BN, 1), lambda i: (i, 0)),
                  _full_spec(64, 64), _full_spec(1, 64), _full_spec(64, 16),
                  _full_spec(1, 16), _full_spec(1, 64), _full_spec(1, 64)],
        out_specs=_full_spec(_GRAPHS, 16),
        out_shape=_SDS((_GRAPHS, 16), _f32),
        scratch_shapes=[pltpu.VMEM((_GRAPHS, 64), _f32)],
        compiler_params=pltpu.CompilerParams(
            dimension_semantics=("arbitrary",)),
    )(z, batch2, fc1_W, fc1br, fc2_W, fc2br, fcngr, fcnbr)

    return out


# speedup vs baseline: 23.9919x; 1.1207x over previous
import functools

import jax
import jax.numpy as jnp
from jax import lax
from jax.experimental import pallas as pl
from jax.experimental.pallas import tpu as pltpu
from jax.experimental.pallas import tpu_sc as plsc

_N = 10000
_NPAD = 10240
_EPAD = 327680
_NC = 2
_NS = 16
_K = 128
_NB = 2
_U = 8
_BN = 1000
_GRAPHS = 64

_f32 = jnp.float32
_SDS = jax.ShapeDtypeStruct
_MESH = dict(core_axis_name="c", subcore_axis_name="s")


def _leaky(v):
    return jnp.where(v >= 0, v, 0.01 * v)


def _make_deg_kernel(npad=_NPAD, epad=_EPAD):
    ept = epad // (_NC * _NS)
    iters8 = ept // (_U * _K)
    rpt = npad // _NS
    mesh = plsc.VectorSubcoreMesh(**_MESH)

    @functools.partial(
        pl.kernel, mesh=mesh,
        out_type=_SDS((_NC, npad), _f32),
        scratch_types=[
            pltpu.VMEM((_U, _K), jnp.int32),
            pltpu.VMEM((_K,), _f32),
            pltpu.VMEM((rpt,), _f32),
            pltpu.VMEM_SHARED((npad,), _f32),
            pltpu.SemaphoreType.DMA,
            pltpu.SemaphoreType.DMA,
        ],
    )
    def deg_kernel(col_h, zeros_h, out_h, cidx8, ones_v, tmp_v, acc, s0, s1):
        c = lax.axis_index("c")
        s = lax.axis_index("s")
        wid = s * _NC + c
        ssems = [s0, s1]
        for j in range(_K // 16):
            ones_v[pl.ds(j * 16, 16)] = jnp.full((16,), 1.0, _f32)
        pltpu.sync_copy(zeros_h, tmp_v)
        pltpu.sync_copy(tmp_v, acc.at[pl.ds(s * rpt, rpt)])
        plsc.subcore_barrier()
        cbase = wid * (ept // _K)

        def outer(i8, carry):
            pltpu.sync_copy(col_h.at[pl.ds(cbase + i8 * _U, _U)], cidx8)
            hs = {}
            for j in range(_U):
                if j >= 2:
                    hs[j - 2].wait()
                hs[j] = pltpu.async_copy(ones_v, acc.at[cidx8.at[j]],
                                         ssems[j % 2], add=True)
            hs[_U - 2].wait()
            hs[_U - 1].wait()
            return carry

        lax.fori_loop(0, iters8, outer, 0)
        plsc.subcore_barrier()
        pltpu.sync_copy(acc.at[pl.ds(s * rpt, rpt)], tmp_v)
        pltpu.sync_copy(tmp_v, out_h.at[c, pl.ds(s * rpt, rpt)])

    return deg_kernel


def _pipelined_loop(gs, row_src, col_src, acc, ridx8, cidx8, bufs, gsems,
                    ssems, cbase, iters8):
    def outer(i8, carry):
        off = cbase + i8 * _U
        pltpu.sync_copy(row_src.at[pl.ds(off, _U)], ridx8)
        pltpu.sync_copy(col_src.at[pl.ds(off, _U)], cidx8)
        hg = {}
        hs = {}
        for j in range(_NB):
            hg[j] = pltpu.async_copy(gs.at[ridx8.at[j]], bufs[j], gsems[j])
        for j in range(_U):
            b = j % _NB
            hg[j].wait()
            hs[j] = pltpu.async_copy(bufs[b], acc.at[cidx8.at[j]], ssems[b],
                                     add=True)
            nj = j + _NB
            if nj < _U:
                hs[j].wait()
                hg[nj] = pltpu.async_copy(gs.at[ridx8.at[nj]], bufs[b],
                                          gsems[b])
        for j in range(_U - _NB, _U):
            hs[j].wait()
        return carry

    lax.fori_loop(0, iters8, outer, 0)


def _make_agg_e(npad=_NPAD, epad=_EPAD, fc=128):
    ept = epad // (_NC * _NS)
    iters8 = ept // (_U * _K)
    rpt = npad // _NS
    rk = min(_K, rpt)
    mesh = plsc.VectorSubcoreMesh(**_MESH)

    @functools.partial(
        pl.kernel, mesh=mesh,
        out_type=_SDS((_NC, npad, fc), _f32),
        scratch_types=[
            pltpu.VMEM((_U, _K), jnp.int32),
            pltpu.VMEM((_U, _K), jnp.int32),
        ] + [pltpu.VMEM((_K, fc), _f32) for _ in range(_NB)] + [
            pltpu.VMEM_SHARED((npad, fc), _f32),
        ] + [pltpu.SemaphoreType.DMA for _ in range(2 * _NB)],
    )
    def agg_kernel(gs, row_h, col_h, zeros_h, out_h,
                   ridx8, cidx8, b0, b1, acc, g0, g1, s0, s1):
        c = lax.axis_index("c")
        s = lax.axis_index("s")
        wid = s * _NC + c
        bufs = [b0, b1]
        gsems = [g0, g1]
        ssems = [s0, s1]
        rows = b0
        pltpu.sync_copy(zeros_h.at[pl.ds(0, rk)], rows.at[pl.ds(0, rk)])
        for j in range(rpt // rk):
            pltpu.sync_copy(rows.at[pl.ds(0, rk)],
                            acc.at[pl.ds(s * rpt + j * rk, rk)])
        plsc.subcore_barrier()
        _pipelined_loop(gs, row_h, col_h, acc, ridx8, cidx8, bufs, gsems,
                        ssems, wid * (ept // _K), iters8)
        plsc.subcore_barrier()
        for j in range(rpt // rk):
            base = s * rpt + j * rk
            pltpu.sync_copy(acc.at[pl.ds(base, rk)], rows.at[pl.ds(0, rk)])
            pltpu.sync_copy(rows.at[pl.ds(0, rk)],
                            out_h.at[c, pl.ds(base, rk)])

    return agg_kernel


def _make_agg_f(npad=_NPAD, epad=_EPAD, fc=128):
    ept = epad // _NS
    iters8 = ept // (_U * _K)
    rpt = npad // _NS
    rk = min(_K, rpt)
    mesh = plsc.VectorSubcoreMesh(**_MESH)

    @functools.partial(
        pl.kernel, mesh=mesh,
        out_type=_SDS((_NC, npad, fc), _f32),
        scratch_types=[
            pltpu.VMEM((_U, _K), jnp.int32),
            pltpu.VMEM((_U, _K), jnp.int32),
        ] + [pltpu.VMEM((_K, fc), _f32) for _ in range(_NB)] + [
            pltpu.VMEM_SHARED((npad, fc), _f32),
        ] + [pltpu.SemaphoreType.DMA for _ in range(2 * _NB)],
    )
    def agg_kernel(gs2n, row2_h, col_h, zeros_h, out_h,
                   ridx8, cidx8, b0, b1, acc, g0, g1, s0, s1):
        c = lax.axis_index("c")
        s = lax.axis_index("s")
        bufs = [b0, b1]
        gsems = [g0, g1]
        ssems = [s0, s1]
        rows = b0
        pltpu.sync_copy(zeros_h.at[pl.ds(0, rk)], rows.at[pl.ds(0, rk)])
        for j in range(rpt // rk):
            pltpu.sync_copy(rows.at[pl.ds(0, rk)],
                            acc.at[pl.ds(s * rpt + j * rk, rk)])
        plsc.subcore_barrier()
        _pipelined_loop(gs2n, row2_h.at[c], col_h, acc, ridx8, cidx8, bufs,
                        gsems, ssems, s * (ept // _K), iters8)
        plsc.subcore_barrier()
        for j in range(rpt // rk):
            base = s * rpt + j * rk
            pltpu.sync_copy(acc.at[pl.ds(base, rk)], rows.at[pl.ds(0, rk)])
            pltpu.sync_copy(rows.at[pl.ds(0, rk)],
                            out_h.at[c, pl.ds(base, rk)])

    return agg_kernel


def _row_spec(w):
    return pl.BlockSpec((_BN, w), lambda i: (i, 0))


def _full_spec(h, w):
    return pl.BlockSpec((h, w), lambda i: (0, 0))


def _plane_spec(p, w):
    return pl.BlockSpec((1, _BN, w), lambda i, _p=p: (_p, i, 0))


def _mm1_body(x, wcat, bres, dcol, gs1, res):
    h = jnp.dot(x[...], wcat[...], preferred_element_type=_f32)
    dinv = lax.rsqrt(dcol[...] + 1.0)
    gs1[...] = jnp.concatenate(
        [dinv * h[:, :64], jnp.zeros((_BN, 64), _f32)], axis=1)
    res[...] = h[:, 64:] + bres[...]


def _ew2_body(gs1, a0, a1, b1, dcol, gs2):
    dinv = lax.rsqrt(dcol[...] + 1.0)
    acc = a0[...][0] + a1[...][0]
    h1 = _leaky(dinv * (acc[:, :64] + gs1[:, :64]) + b1[...])
    gs2[...] = jnp.concatenate(
        [dinv * h1, jnp.zeros((_BN, 64), _f32)], axis=1)


def _mm2_body(gs2, a0, a1, w2, b2, dcol, gs3):
    dinv = lax.rsqrt(dcol[...] + 1.0)
    acc = a0[...][0] + a1[...][0]
    a2 = dinv * (acc[:, :64] + gs2[:, :64])
    h2 = _leaky(jnp.dot(a2, w2[...], preferred_element_type=_f32) + b2[...])
    g = dinv * h2
    gs3[...] = jnp.stack([g[:, :128], g[:, 128:]], axis=0)


def _mm34_body(gs3, f0, f1, w3, b3, w4, dcol, gs4):
    dinv = lax.rsqrt(dcol[...] + 1.0)
    g3 = gs3[...]
    a3 = dinv * jnp.concatenate(
        [f0[...][0] + g3[0], f1[...][0] + g3[1]], axis=1)
    h3 = _leaky(jnp.dot(a3, w3[...], preferred_element_type=_f32) + b3[...])
    m4 = jnp.dot(h3, w4[...], preferred_element_type=_f32)
    gs4[...] = jnp.concatenate(
        [dinv * m4, jnp.zeros((_BN, 64), _f32)], axis=1)


def _fin_body(gs4, a0, a1, b4, resi, lng, lnb, dcol, z):
    dinv = lax.rsqrt(dcol[...] + 1.0)
    acc = a0[...][0] + a1[...][0]
    h4 = _leaky(dinv * (acc[:, :64] + gs4[:, :64]) + b4[...])
    y = h4 + resi[...]
    m = jnp.mean(y, axis=-1, keepdims=True)
    v = jnp.mean((y - m) * (y - m), axis=-1, keepdims=True)
    z[...] = (y - m) * lax.rsqrt(v + 1e-5) * lng[...] + lnb[...]


def _pool_head_body(zb, bb, fc1w, fc1b, fc2w, fc2b, fcng, fcnb, out,
                    pool_acc):
    i = pl.program_id(0)
    nsteps = pl.num_programs(0)

    @pl.when(i == 0)
    def _():
        pool_acc[...] = jnp.full((_GRAPHS, 64), -jnp.inf, _f32)

    zv = zb[...]
    bv = bb[...]
    g_lo = bb[0, 0]
    g_hi = bb[_BN - 1, 0]
    for g in range(_GRAPHS):
        @pl.when((g_lo <= g) & (g <= g_hi))
        def _():
            mask = bv == g
            rowmax = jnp.max(jnp.where(mask, zv, -jnp.inf), axis=0,
                             keepdims=True)
            pool_acc[pl.ds(g, 1), :] = jnp.maximum(
                pool_acc[pl.ds(g, 1), :], rowmax)

    @pl.when(i == nsteps - 1)
    def _():
        p = pool_acc[...]
        h = jnp.dot(p, fc1w[...], preferred_element_type=_f32) + fc1b[...]
        m = jnp.mean(h, axis=-1, keepdims=True)
        v = jnp.mean((h - m) * (h - m), axis=-1, keepdims=True)
        h = (h - m) * lax.rsqrt(v + 1e-5) * fcng[...] + fcnb[...]
        h = _leaky(h)
        out[...] = jnp.dot(h, fc2w[...], preferred_element_type=_f32) \
            + fc2b[...]


def kernel(x, edge_index, batch, W1, b1, W2, b2, W3, b3, W4, b4, Wres, bres,
           ln_g, ln_b, fcn_g, fcn_b, fc1_W, fc1_b, fc2_W, fc2_b):
    n = x.shape[0]
    e = edge_index.shape[1]
    grid = (n // _BN,)

    pad = _EPAD - e
    ar = jnp.arange(pad, dtype=jnp.int32)
    row_pad = jnp.concatenate([edge_index[0], ar % n])
    col_pad = jnp.concatenate([edge_index[1], n + (ar % (_NPAD - n))])
    row2 = jnp.stack([row_pad, row_pad + jnp.int32(n)])
    zeros1 = jnp.zeros((_NPAD // _NS,), _f32)
    zeros128 = jnp.zeros((_K, 128), _f32)
    wcat = jnp.concatenate([W1, Wres], axis=1)
    b1r = b1.reshape(1, 64)
    b2r = b2.reshape(1, 256)
    b3r = b3.reshape(1, 512)
    b4r = b4.reshape(1, 64)
    bresr = bres.reshape(1, 64)
    lngr = ln_g.reshape(1, 64)
    lnbr = ln_b.reshape(1, 64)
    fcngr = fcn_g.reshape(1, 64)
    fcnbr = fcn_b.reshape(1, 64)
    fc1br = fc1_b.reshape(1, 64)
    fc2br = fc2_b.reshape(1, 16)
    batch2 = batch.reshape(n, 1)

    row2d = row_pad.reshape(_EPAD // _K, _K)
    col2d = col_pad.reshape(_EPAD // _K, _K)
    row23d = row2.reshape(2, _EPAD // _K, _K)

    degp = _make_deg_kernel()(col2d, zeros1)
    dcol = (degp[0, :n] + degp[1, :n]).reshape(n, 1)

    dspec = _row_spec(1)
    agg64 = _make_agg_e()
    p0s = _plane_spec(0, 128)
    p1s = _plane_spec(1, 128)

    gs1, res = pl.pallas_call(
        _mm1_body,
        grid=grid,
        in_specs=[_row_spec(128), _full_spec(128, 128), _full_spec(1, 64),
                  dspec],
        out_specs=[_row_spec(128), _row_spec(64)],
        out_shape=[_SDS((n, 128), _f32), _SDS((n, 64), _f32)],
    )(x, wcat, bresr, dcol)

    acc1 = agg64(gs1, row2d, col2d, zeros128)

    gs2 = pl.pallas_call(
        _ew2_body,
        grid=grid,
        in_specs=[_row_spec(128), p0s, p1s, _full_spec(1, 64), dspec],
        out_specs=_row_spec(128),
        out_shape=_SDS((n, 128), _f32),
    )(gs1, acc1, acc1, b1r, dcol)

    acc2 = agg64(gs2, row2d, col2d, zeros128)

    gs3 = pl.pallas_call(
        _mm2_body,
        grid=grid,
        in_specs=[_row_spec(128), p0s, p1s, _full_spec(64, 256),
                  _full_spec(1, 256), dspec],
        out_specs=pl.BlockSpec((2, _BN, 128), lambda i: (0, i, 0)),
        out_shape=_SDS((2, n, 128), _f32),
    )(gs2, acc2, acc2, W2, b2r, dcol)

    gs3cat = gs3.reshape(2 * n, 128)
    acc3 = _make_agg_f()(gs3cat, row23d, col2d, zeros128)

    gs4 = pl.pallas_call(
        _mm34_body,
        grid=grid,
        in_specs=[pl.BlockSpec((2, _BN, 128), lambda i: (0, i, 0)),
                  p0s, p1s, _full_spec(256, 512), _full_spec(1, 512),
                  _full_spec(512, 64), dspec],
        out_specs=_row_spec(128),
        out_shape=_SDS((n, 128), _f32),
    )(gs3, acc3, acc3, W3, b3r, W4, dcol)

    acc4 = agg64(gs4, row2d, col2d, zeros128)

    z = pl.pallas_call(
        _fin_body,
        grid=grid,
        in_specs=[_row_spec(128), p0s, p1s, _full_spec(1, 64), _row_spec(64),
                  _full_spec(1, 64), _full_spec(1, 64), dspec],
        out_specs=_row_spec(64),
        out_shape=_SDS((n, 64), _f32),
    )(gs4, acc4, acc4, b4r, res, lngr, lnbr, dcol)

    out = pl.pallas_call(
        _pool_head_body,
        grid=grid,
        in_specs=[_row_spec(64), pl.BlockSpec((_BN, 1), lambda i: (i, 0)),
                  _full_spec(64, 64), _full_spec(1, 64), _full_spec(64, 16),
                  _full_spec(1, 16), _full_spec(1, 64), _full_spec(1, 64)],
        out_specs=_full_spec(_GRAPHS, 16),
        out_shape=_SDS((_GRAPHS, 16), _f32),
        scratch_shapes=[pltpu.VMEM((_GRAPHS, 64), _f32)],
        compiler_params=pltpu.CompilerParams(
            dimension_semantics=("arbitrary",)),
    )(z, batch2, fc1_W, fc1br, fc2_W, fc2br, fcngr, fcnbr)

    return out


# speedup vs baseline: 26.2442x; 1.0939x over previous
import functools

import jax
import jax.numpy as jnp
from jax import lax
from jax.experimental import pallas as pl
from jax.experimental.pallas import tpu as pltpu
from jax.experimental.pallas import tpu_sc as plsc

_N = 10000
_NPAD = 10240
_EPAD = 327680
_NC = 2
_NS = 16
_K = 128
_NB = 2
_U = 16
_BN = 1000
_GRAPHS = 64

_f32 = jnp.float32
_SDS = jax.ShapeDtypeStruct
_MESH = dict(core_axis_name="c", subcore_axis_name="s")


def _leaky(v):
    return jnp.where(v >= 0, v, 0.01 * v)


def _make_deg_kernel(npad=_NPAD, epad=_EPAD):
    ept = epad // (_NC * _NS)
    iters8 = ept // (_U * _K)
    rpt = npad // _NS
    mesh = plsc.VectorSubcoreMesh(**_MESH)

    @functools.partial(
        pl.kernel, mesh=mesh,
        out_type=_SDS((_NC, npad), _f32),
        scratch_types=[
            pltpu.VMEM((_U, _K), jnp.int32),
            pltpu.VMEM((_K,), _f32),
            pltpu.VMEM((rpt,), _f32),
            pltpu.VMEM_SHARED((npad,), _f32),
            pltpu.SemaphoreType.DMA,
            pltpu.SemaphoreType.DMA,
        ],
    )
    def deg_kernel(col_h, zeros_h, out_h, cidx8, ones_v, tmp_v, acc, s0, s1):
        c = lax.axis_index("c")
        s = lax.axis_index("s")
        wid = s * _NC + c
        ssems = [s0, s1]
        for j in range(_K // 16):
            ones_v[pl.ds(j * 16, 16)] = jnp.full((16,), 1.0, _f32)
        pltpu.sync_copy(zeros_h, tmp_v)
        pltpu.sync_copy(tmp_v, acc.at[pl.ds(s * rpt, rpt)])
        plsc.subcore_barrier()
        cbase = wid * (ept // _K)

        def outer(i8, carry):
            pltpu.sync_copy(col_h.at[pl.ds(cbase + i8 * _U, _U)], cidx8)
            hs = {}
            for j in range(_U):
                if j >= 2:
                    hs[j - 2].wait()
                hs[j] = pltpu.async_copy(ones_v, acc.at[cidx8.at[j]],
                                         ssems[j % 2], add=True)
            hs[_U - 2].wait()
            hs[_U - 1].wait()
            return carry

        lax.fori_loop(0, iters8, outer, 0)
        plsc.subcore_barrier()
        pltpu.sync_copy(acc.at[pl.ds(s * rpt, rpt)], tmp_v)
        pltpu.sync_copy(tmp_v, out_h.at[c, pl.ds(s * rpt, rpt)])

    return deg_kernel


def _pipelined_loop(gs, row_src, col_src, acc, ridx8, cidx8, bufs, gsems,
                    ssems, cbase, iters8):
    def outer(i8, carry):
        off = cbase + i8 * _U
        pltpu.sync_copy(row_src.at[pl.ds(off, _U)], ridx8)
        pltpu.sync_copy(col_src.at[pl.ds(off, _U)], cidx8)
        hg = {}
        hs = {}
        for j in range(_NB):
            hg[j] = pltpu.async_copy(gs.at[ridx8.at[j]], bufs[j], gsems[j])
        for j in range(_U):
            b = j % _NB
            hg[j].wait()
            hs[j] = pltpu.async_copy(bufs[b], acc.at[cidx8.at[j]], ssems[b],
                                     add=True)
            nj = j + _NB
            if nj < _U:
                hs[j].wait()
                hg[nj] = pltpu.async_copy(gs.at[ridx8.at[nj]], bufs[b],
                                          gsems[b])
        for j in range(_U - _NB, _U):
            hs[j].wait()
        return carry

    lax.fori_loop(0, iters8, outer, 0)


def _make_agg_e(npad=_NPAD, epad=_EPAD, fc=128):
    ept = epad // (_NC * _NS)
    iters8 = ept // (_U * _K)
    rpt = npad // _NS
    rk = min(_K, rpt)
    mesh = plsc.VectorSubcoreMesh(**_MESH)

    @functools.partial(
        pl.kernel, mesh=mesh,
        out_type=_SDS((_NC, npad, fc), _f32),
        scratch_types=[
            pltpu.VMEM((_U, _K), jnp.int32),
            pltpu.VMEM((_U, _K), jnp.int32),
        ] + [pltpu.VMEM((_K, fc), _f32) for _ in range(_NB)] + [
            pltpu.VMEM_SHARED((npad, fc), _f32),
        ] + [pltpu.SemaphoreType.DMA for _ in range(2 * _NB)],
    )
    def agg_kernel(gs, row_h, col_h, zeros_h, out_h,
                   ridx8, cidx8, b0, b1, acc, g0, g1, s0, s1):
        c = lax.axis_index("c")
        s = lax.axis_index("s")
        wid = s * _NC + c
        bufs = [b0, b1]
        gsems = [g0, g1]
        ssems = [s0, s1]
        rows = b0
        pltpu.sync_copy(zeros_h.at[pl.ds(0, rk)], rows.at[pl.ds(0, rk)])
        for j in range(rpt // rk):
            pltpu.sync_copy(rows.at[pl.ds(0, rk)],
                            acc.at[pl.ds(s * rpt + j * rk, rk)])
        plsc.subcore_barrier()
        _pipelined_loop(gs, row_h, col_h, acc, ridx8, cidx8, bufs, gsems,
                        ssems, wid * (ept // _K), iters8)
        plsc.subcore_barrier()
        for j in range(rpt // rk):
            base = s * rpt + j * rk
            pltpu.sync_copy(acc.at[pl.ds(base, rk)], rows.at[pl.ds(0, rk)])
            pltpu.sync_copy(rows.at[pl.ds(0, rk)],
                            out_h.at[c, pl.ds(base, rk)])

    return agg_kernel


def _make_agg_f(npad=_NPAD, epad=_EPAD, fc=128):
    ept = epad // _NS
    iters8 = ept // (_U * _K)
    rpt = npad // _NS
    rk = min(_K, rpt)
    mesh = plsc.VectorSubcoreMesh(**_MESH)

    @functools.partial(
        pl.kernel, mesh=mesh,
        out_type=_SDS((_NC, npad, fc), _f32),
        scratch_types=[
            pltpu.VMEM((_U, _K), jnp.int32),
            pltpu.VMEM((_U, _K), jnp.int32),
        ] + [pltpu.VMEM((_K, fc), _f32) for _ in range(_NB)] + [
            pltpu.VMEM_SHARED((npad, fc), _f32),
        ] + [pltpu.SemaphoreType.DMA for _ in range(2 * _NB)],
    )
    def agg_kernel(gs2n, row2_h, col_h, zeros_h, out_h,
                   ridx8, cidx8, b0, b1, acc, g0, g1, s0, s1):
        c = lax.axis_index("c")
        s = lax.axis_index("s")
        bufs = [b0, b1]
        gsems = [g0, g1]
        ssems = [s0, s1]
        rows = b0
        pltpu.sync_copy(zeros_h.at[pl.ds(0, rk)], rows.at[pl.ds(0, rk)])
        for j in range(rpt // rk):
            pltpu.sync_copy(rows.at[pl.ds(0, rk)],
                            acc.at[pl.ds(s * rpt + j * rk, rk)])
        plsc.subcore_barrier()
        _pipelined_loop(gs2n, row2_h.at[c], col_h, acc, ridx8, cidx8, bufs,
                        gsems, ssems, s * (ept // _K), iters8)
        plsc.subcore_barrier()
        for j in range(rpt // rk):
            base = s * rpt + j * rk
            pltpu.sync_copy(acc.at[pl.ds(base, rk)], rows.at[pl.ds(0, rk)])
            pltpu.sync_copy(rows.at[pl.ds(0, rk)],
                            out_h.at[c, pl.ds(base, rk)])

    return agg_kernel


def _row_spec(w):
    return pl.BlockSpec((_BN, w), lambda i: (i, 0))


def _full_spec(h, w):
    return pl.BlockSpec((h, w), lambda i: (0, 0))


def _plane_spec(p, w):
    return pl.BlockSpec((1, _BN, w), lambda i, _p=p: (_p, i, 0))


def _mm1_body(x, wcat, bres, dcol, gs1, res):
    h = jnp.dot(x[...], wcat[...], preferred_element_type=_f32)
    dinv = lax.rsqrt(dcol[...] + 1.0)
    gs1[...] = jnp.concatenate(
        [dinv * h[:, :64], jnp.zeros((_BN, 64), _f32)], axis=1)
    res[...] = h[:, 64:] + bres[...]


def _ew2_body(gs1, a0, a1, b1, dcol, gs2):
    dinv = lax.rsqrt(dcol[...] + 1.0)
    acc = a0[...][0] + a1[...][0]
    h1 = _leaky(dinv * (acc[:, :64] + gs1[:, :64]) + b1[...])
    gs2[...] = jnp.concatenate(
        [dinv * h1, jnp.zeros((_BN, 64), _f32)], axis=1)


def _mm2_body(gs2, a0, a1, w2, b2, dcol, gs3):
    dinv = lax.rsqrt(dcol[...] + 1.0)
    acc = a0[...][0] + a1[...][0]
    a2 = dinv * (acc[:, :64] + gs2[:, :64])
    h2 = _leaky(jnp.dot(a2, w2[...], preferred_element_type=_f32) + b2[...])
    g = dinv * h2
    gs3[...] = jnp.stack([g[:, :128], g[:, 128:]], axis=0)


def _mm34_body(gs3, f0, f1, w3, b3, w4, dcol, gs4):
    dinv = lax.rsqrt(dcol[...] + 1.0)
    g3 = gs3[...]
    a3 = dinv * jnp.concatenate(
        [f0[...][0] + g3[0], f1[...][0] + g3[1]], axis=1)
    h3 = _leaky(jnp.dot(a3, w3[...], preferred_element_type=_f32) + b3[...])
    m4 = jnp.dot(h3, w4[...], preferred_element_type=_f32)
    gs4[...] = jnp.concatenate(
        [dinv * m4, jnp.zeros((_BN, 64), _f32)], axis=1)


def _fin_pool_head_body(gs4, a0, a1, b4, resi, lng, lnb, dcol, bb,
                        fc1w, fc1b, fc2w, fc2b, fcng, fcnb, out, pool_acc):
    i = pl.program_id(0)
    nsteps = pl.num_programs(0)

    dinv = lax.rsqrt(dcol[...] + 1.0)
    acc = a0[...][0] + a1[...][0]
    h4 = _leaky(dinv * (acc[:, :64] + gs4[:, :64]) + b4[...])
    y = h4 + resi[...]
    m = jnp.mean(y, axis=-1, keepdims=True)
    v = jnp.mean((y - m) * (y - m), axis=-1, keepdims=True)
    zv = (y - m) * lax.rsqrt(v + 1e-5) * lng[...] + lnb[...]

    @pl.when(i == 0)
    def _():
        pool_acc[...] = jnp.full((_GRAPHS, 64), -jnp.inf, _f32)

    bv = bb[...]
    g_lo = bb[0, 0]
    g_hi = bb[_BN - 1, 0]
    for g in range(_GRAPHS):
        @pl.when((g_lo <= g) & (g <= g_hi))
        def _():
            mask = bv == g
            rowmax = jnp.max(jnp.where(mask, zv, -jnp.inf), axis=0,
                             keepdims=True)
            pool_acc[pl.ds(g, 1), :] = jnp.maximum(
                pool_acc[pl.ds(g, 1), :], rowmax)

    @pl.when(i == nsteps - 1)
    def _():
        p = pool_acc[...]
        h = jnp.dot(p, fc1w[...], preferred_element_type=_f32) + fc1b[...]
        m2 = jnp.mean(h, axis=-1, keepdims=True)
        v2 = jnp.mean((h - m2) * (h - m2), axis=-1, keepdims=True)
        h = (h - m2) * lax.rsqrt(v2 + 1e-5) * fcng[...] + fcnb[...]
        h = _leaky(h)
        out[...] = jnp.dot(h, fc2w[...], preferred_element_type=_f32) \
            + fc2b[...]


def kernel(x, edge_index, batch, W1, b1, W2, b2, W3, b3, W4, b4, Wres, bres,
           ln_g, ln_b, fcn_g, fcn_b, fc1_W, fc1_b, fc2_W, fc2_b):
    n = x.shape[0]
    e = edge_index.shape[1]
    grid = (n // _BN,)

    pad = _EPAD - e
    ar = jnp.arange(pad, dtype=jnp.int32)
    row_pad = jnp.concatenate([edge_index[0], ar % n])
    col_pad = jnp.concatenate([edge_index[1], n + (ar % (_NPAD - n))])
    row2 = jnp.stack([row_pad, row_pad + jnp.int32(n)])
    zeros1 = jnp.zeros((_NPAD // _NS,), _f32)
    zeros128 = jnp.zeros((_K, 128), _f32)
    wcat = jnp.concatenate([W1, Wres], axis=1)
    b1r = b1.reshape(1, 64)
    b2r = b2.reshape(1, 256)
    b3r = b3.reshape(1, 512)
    b4r = b4.reshape(1, 64)
    bresr = bres.reshape(1, 64)
    lngr = ln_g.reshape(1, 64)
    lnbr = ln_b.reshape(1, 64)
    fcngr = fcn_g.reshape(1, 64)
    fcnbr = fcn_b.reshape(1, 64)
    fc1br = fc1_b.reshape(1, 64)
    fc2br = fc2_b.reshape(1, 16)
    batch2 = batch.reshape(n, 1)

    row2d = row_pad.reshape(_EPAD // _K, _K)
    col2d = col_pad.reshape(_EPAD // _K, _K)
    row23d = row2.reshape(2, _EPAD // _K, _K)

    degp = _make_deg_kernel()(col2d, zeros1)
    dcol = (degp[0, :n] + degp[1, :n]).reshape(n, 1)

    dspec = _row_spec(1)
    agg64 = _make_agg_e()
    p0s = _plane_spec(0, 128)
    p1s = _plane_spec(1, 128)
    f0s = p0s
    f1s = p1s

    gs1, res = pl.pallas_call(
        _mm1_body,
        grid=grid,
        in_specs=[_row_spec(128), _full_spec(128, 128), _full_spec(1, 64),
                  dspec],
        out_specs=[_row_spec(128), _row_spec(64)],
        out_shape=[_SDS((n, 128), _f32), _SDS((n, 64), _f32)],
    )(x, wcat, bresr, dcol)

    acc1 = agg64(gs1, row2d, col2d, zeros128)

    gs2 = pl.pallas_call(
        _ew2_body,
        grid=grid,
        in_specs=[_row_spec(128), p0s, p1s, _full_spec(1, 64), dspec],
        out_specs=_row_spec(128),
        out_shape=_SDS((n, 128), _f32),
    )(gs1, acc1, acc1, b1r, dcol)

    acc2 = agg64(gs2, row2d, col2d, zeros128)

    gs3 = pl.pallas_call(
        _mm2_body,
        grid=grid,
        in_specs=[_row_spec(128), p0s, p1s, _full_spec(64, 256),
                  _full_spec(1, 256), dspec],
        out_specs=pl.BlockSpec((2, _BN, 128), lambda i: (0, i, 0)),
        out_shape=_SDS((2, n, 128), _f32),
    )(gs2, acc2, acc2, W2, b2r, dcol)

    gs3cat = gs3.reshape(2 * n, 128)
    acc3 = _make_agg_f()(gs3cat, row23d, col2d, zeros128)

    gs4 = pl.pallas_call(
        _mm34_body,
        grid=grid,
        in_specs=[pl.BlockSpec((2, _BN, 128), lambda i: (0, i, 0)),
                  f0s, f1s, _full_spec(256, 512), _full_spec(1, 512),
                  _full_spec(512, 64), dspec],
        out_specs=_row_spec(128),
        out_shape=_SDS((n, 128), _f32),
    )(gs3, acc3, acc3, W3, b3r, W4, dcol)

    acc4 = agg64(gs4, row2d, col2d, zeros128)

    out = pl.pallas_call(
        _fin_pool_head_body,
        grid=grid,
        in_specs=[_row_spec(128), p0s, p1s, _full_spec(1, 64), _row_spec(64),
                  _full_spec(1, 64), _full_spec(1, 64), dspec,
                  pl.BlockSpec((_BN, 1), lambda i: (i, 0)),
                  _full_spec(64, 64), _full_spec(1, 64), _full_spec(64, 16),
                  _full_spec(1, 16), _full_spec(1, 64), _full_spec(1, 64)],
        out_specs=_full_spec(_GRAPHS, 16),
        out_shape=_SDS((_GRAPHS, 16), _f32),
        scratch_shapes=[pltpu.VMEM((_GRAPHS, 64), _f32)],
        compiler_params=pltpu.CompilerParams(
            dimension_semantics=("arbitrary",)),
    )(gs4, acc4, acc4, b4r, res, lngr, lnbr, dcol, batch2,
      fc1_W, fc1br, fc2_W, fc2br, fcngr, fcnbr)

    return out


# speedup vs baseline: 26.5943x; 1.0133x over previous
import functools

import jax
import jax.numpy as jnp
from jax import lax
from jax.experimental import pallas as pl
from jax.experimental.pallas import tpu as pltpu
from jax.experimental.pallas import tpu_sc as plsc

_N = 10000
_NPAD = 10240
_EPAD = 327680
_NC = 2
_NS = 16
_K = 80
_NB = 4
_U = 16
_BN = 1000
_GRAPHS = 64

_f32 = jnp.float32
_SDS = jax.ShapeDtypeStruct
_MESH = dict(core_axis_name="c", subcore_axis_name="s")


def _leaky(v):
    return jnp.where(v >= 0, v, 0.01 * v)


def _make_deg_kernel(npad=_NPAD, epad=_EPAD):
    ept = epad // (_NC * _NS)
    iters8 = ept // (_U * _K)
    rpt = npad // _NS
    mesh = plsc.VectorSubcoreMesh(**_MESH)

    @functools.partial(
        pl.kernel, mesh=mesh,
        out_type=_SDS((_NC, npad), _f32),
        scratch_types=[
            pltpu.VMEM((_U, _K), jnp.int32),
            pltpu.VMEM((_K,), _f32),
            pltpu.VMEM((rpt,), _f32),
            pltpu.VMEM_SHARED((npad,), _f32),
            pltpu.SemaphoreType.DMA,
            pltpu.SemaphoreType.DMA,
        ],
    )
    def deg_kernel(col_h, zeros_h, out_h, cidx8, ones_v, tmp_v, acc, s0, s1):
        c = lax.axis_index("c")
        s = lax.axis_index("s")
        wid = s * _NC + c
        ssems = [s0, s1]
        for j in range(_K // 16):
            ones_v[pl.ds(j * 16, 16)] = jnp.full((16,), 1.0, _f32)
        pltpu.sync_copy(zeros_h, tmp_v)
        pltpu.sync_copy(tmp_v, acc.at[pl.ds(s * rpt, rpt)])
        plsc.subcore_barrier()
        cbase = wid * (ept // _K)

        def outer(i8, carry):
            pltpu.sync_copy(col_h.at[pl.ds(cbase + i8 * _U, _U)], cidx8)
            hs = {}
            for j in range(_U):
                if j >= 2:
                    hs[j - 2].wait()
                hs[j] = pltpu.async_copy(ones_v, acc.at[cidx8.at[j]],
                                         ssems[j % 2], add=True)
            hs[_U - 2].wait()
            hs[_U - 1].wait()
            return carry

        lax.fori_loop(0, iters8, outer, 0)
        plsc.subcore_barrier()
        pltpu.sync_copy(acc.at[pl.ds(s * rpt, rpt)], tmp_v)
        pltpu.sync_copy(tmp_v, out_h.at[c, pl.ds(s * rpt, rpt)])

    return deg_kernel


def _pipelined_loop(gs, row_src, col_src, acc, ridx8, cidx8, bufs, gsems,
                    ssems, cbase, iters8):
    def outer(i8, carry):
        off = cbase + i8 * _U
        pltpu.sync_copy(row_src.at[pl.ds(off, _U)], ridx8)
        pltpu.sync_copy(col_src.at[pl.ds(off, _U)], cidx8)
        hg = {}
        hs = {}
        for j in range(_NB):
            hg[j] = pltpu.async_copy(gs.at[ridx8.at[j]], bufs[j], gsems[j])
        for j in range(_U):
            b = j % _NB
            hg[j].wait()
            hs[j] = pltpu.async_copy(bufs[b], acc.at[cidx8.at[j]], ssems[b],
                                     add=True)
            nj = j + _NB
            if nj < _U:
                hs[j].wait()
                hg[nj] = pltpu.async_copy(gs.at[ridx8.at[nj]], bufs[b],
                                          gsems[b])
        for j in range(_U - _NB, _U):
            hs[j].wait()
        return carry

    lax.fori_loop(0, iters8, outer, 0)


def _make_agg_e(npad=_NPAD, epad=_EPAD, fc=128):
    ept = epad // (_NC * _NS)
    iters8 = ept // (_U * _K)
    rpt = npad // _NS
    rk = min(_K, rpt)
    mesh = plsc.VectorSubcoreMesh(**_MESH)

    @functools.partial(
        pl.kernel, mesh=mesh,
        out_type=_SDS((_NC, npad, fc), _f32),
        scratch_types=[
            pltpu.VMEM((_U, _K), jnp.int32),
            pltpu.VMEM((_U, _K), jnp.int32),
        ] + [pltpu.VMEM((_K, fc), _f32) for _ in range(_NB)] + [
            pltpu.VMEM_SHARED((npad, fc), _f32),
        ] + [pltpu.SemaphoreType.DMA for _ in range(2 * _NB)],
    )
    def agg_kernel(gs, row_h, col_h, zeros_h, out_h,
                   ridx8, cidx8, b0, b1, b2, b3, acc,
                   g0, g1, g2, g3, s0, s1, s2, s3):
        c = lax.axis_index("c")
        s = lax.axis_index("s")
        wid = s * _NC + c
        bufs = [b0, b1, b2, b3]
        gsems = [g0, g1, g2, g3]
        ssems = [s0, s1, s2, s3]
        rows = b0
        pltpu.sync_copy(zeros_h.at[pl.ds(0, rk)], rows.at[pl.ds(0, rk)])
        for j in range(rpt // rk):
            pltpu.sync_copy(rows.at[pl.ds(0, rk)],
                            acc.at[pl.ds(s * rpt + j * rk, rk)])
        plsc.subcore_barrier()
        _pipelined_loop(gs, row_h, col_h, acc, ridx8, cidx8, bufs, gsems,
                        ssems, wid * (ept // _K), iters8)
        plsc.subcore_barrier()
        for j in range(rpt // rk):
            base = s * rpt + j * rk
            pltpu.sync_copy(acc.at[pl.ds(base, rk)], rows.at[pl.ds(0, rk)])
            pltpu.sync_copy(rows.at[pl.ds(0, rk)],
                            out_h.at[c, pl.ds(base, rk)])

    return agg_kernel


def _make_agg_f(npad=_NPAD, epad=_EPAD, fc=128):
    ept = epad // _NS
    iters8 = ept // (_U * _K)
    rpt = npad // _NS
    rk = min(_K, rpt)
    mesh = plsc.VectorSubcoreMesh(**_MESH)

    @functools.partial(
        pl.kernel, mesh=mesh,
        out_type=_SDS((_NC, npad, fc), _f32),
        scratch_types=[
            pltpu.VMEM((_U, _K), jnp.int32),
            pltpu.VMEM((_U, _K), jnp.int32),
        ] + [pltpu.VMEM((_K, fc), _f32) for _ in range(_NB)] + [
            pltpu.VMEM_SHARED((npad, fc), _f32),
        ] + [pltpu.SemaphoreType.DMA for _ in range(2 * _NB)],
    )
    def agg_kernel(gs2n, row2_h, col_h, zeros_h, out_h,
                   ridx8, cidx8, b0, b1, b2, b3, acc,
                   g0, g1, g2, g3, s0, s1, s2, s3):
        c = lax.axis_index("c")
        s = lax.axis_index("s")
        bufs = [b0, b1, b2, b3]
        gsems = [g0, g1, g2, g3]
        ssems = [s0, s1, s2, s3]
        rows = b0
        pltpu.sync_copy(zeros_h.at[pl.ds(0, rk)], rows.at[pl.ds(0, rk)])
        for j in range(rpt // rk):
            pltpu.sync_copy(rows.at[pl.ds(0, rk)],
                            acc.at[pl.ds(s * rpt + j * rk, rk)])
        plsc.subcore_barrier()
        _pipelined_loop(gs2n, row2_h.at[c], col_h, acc, ridx8, cidx8, bufs,
                        gsems, ssems, s * (ept // _K), iters8)
        plsc.subcore_barrier()
        for j in range(rpt // rk):
            base = s * rpt + j * rk
            pltpu.sync_copy(acc.at[pl.ds(base, rk)], rows.at[pl.ds(0, rk)])
            pltpu.sync_copy(rows.at[pl.ds(0, rk)],
                            out_h.at[c, pl.ds(base, rk)])

    return agg_kernel


def _row_spec(w):
    return pl.BlockSpec((_BN, w), lambda i: (i, 0))


def _full_spec(h, w):
    return pl.BlockSpec((h, w), lambda i: (0, 0))


def _plane_spec(p, w):
    return pl.BlockSpec((1, _BN, w), lambda i, _p=p: (_p, i, 0))


def _mm1_body(x, wcat, bres, dcol, gs1, res):
    h = jnp.dot(x[...], wcat[...], preferred_element_type=_f32)
    dinv = lax.rsqrt(dcol[...] + 1.0)
    gs1[...] = jnp.concatenate(
        [dinv * h[:, :64], jnp.zeros((_BN, 64), _f32)], axis=1)
    res[...] = h[:, 64:] + bres[...]


def _ew2_body(gs1, a0, a1, b1, dcol, gs2):
    dinv = lax.rsqrt(dcol[...] + 1.0)
    acc = a0[...][0] + a1[...][0]
    h1 = _leaky(dinv * (acc[:, :64] + gs1[:, :64]) + b1[...])
    gs2[...] = jnp.concatenate(
        [dinv * h1, jnp.zeros((_BN, 64), _f32)], axis=1)


def _mm2_body(gs2, a0, a1, w2, b2, dcol, gs3):
    dinv = lax.rsqrt(dcol[...] + 1.0)
    acc = a0[...][0] + a1[...][0]
    a2 = dinv * (acc[:, :64] + gs2[:, :64])
    h2 = _leaky(jnp.dot(a2, w2[...], preferred_element_type=_f32) + b2[...])
    g = dinv * h2
    gs3[...] = jnp.stack([g[:, :128], g[:, 128:]], axis=0)


def _mm34_body(gs3, f0, f1, w3, b3, w4, dcol, gs4):
    dinv = lax.rsqrt(dcol[...] + 1.0)
    g3 = gs3[...]
    a3 = dinv * jnp.concatenate(
        [f0[...][0] + g3[0], f1[...][0] + g3[1]], axis=1)
    h3 = _leaky(jnp.dot(a3, w3[...], preferred_element_type=_f32) + b3[...])
    m4 = jnp.dot(h3, w4[...], preferred_element_type=_f32)
    gs4[...] = jnp.concatenate(
        [dinv * m4, jnp.zeros((_BN, 64), _f32)], axis=1)


def _fin_pool_head_body(gs4, a0, a1, b4, resi, lng, lnb, dcol, bb,
                        fc1w, fc1b, fc2w, fc2b, fcng, fcnb, out, pool_acc):
    i = pl.program_id(0)
    nsteps = pl.num_programs(0)

    dinv = lax.rsqrt(dcol[...] + 1.0)
    acc = a0[...][0] + a1[...][0]
    h4 = _leaky(dinv * (acc[:, :64] + gs4[:, :64]) + b4[...])
    y = h4 + resi[...]
    m = jnp.mean(y, axis=-1, keepdims=True)
    v = jnp.mean((y - m) * (y - m), axis=-1, keepdims=True)
    zv = (y - m) * lax.rsqrt(v + 1e-5) * lng[...] + lnb[...]

    @pl.when(i == 0)
    def _():
        pool_acc[...] = jnp.full((_GRAPHS, 64), -jnp.inf, _f32)

    bv = bb[...]
    g_lo = bb[0, 0]
    g_hi = bb[_BN - 1, 0]
    for g in range(_GRAPHS):
        @pl.when((g_lo <= g) & (g <= g_hi))
        def _():
            mask = bv == g
            rowmax = jnp.max(jnp.where(mask, zv, -jnp.inf), axis=0,
                             keepdims=True)
            pool_acc[pl.ds(g, 1), :] = jnp.maximum(
                pool_acc[pl.ds(g, 1), :], rowmax)

    @pl.when(i == nsteps - 1)
    def _():
        p = pool_acc[...]
        h = jnp.dot(p, fc1w[...], preferred_element_type=_f32) + fc1b[...]
        m2 = jnp.mean(h, axis=-1, keepdims=True)
        v2 = jnp.mean((h - m2) * (h - m2), axis=-1, keepdims=True)
        h = (h - m2) * lax.rsqrt(v2 + 1e-5) * fcng[...] + fcnb[...]
        h = _leaky(h)
        out[...] = jnp.dot(h, fc2w[...], preferred_element_type=_f32) \
            + fc2b[...]


def kernel(x, edge_index, batch, W1, b1, W2, b2, W3, b3, W4, b4, Wres, bres,
           ln_g, ln_b, fcn_g, fcn_b, fc1_W, fc1_b, fc2_W, fc2_b):
    n = x.shape[0]
    e = edge_index.shape[1]
    grid = (n // _BN,)

    pad = _EPAD - e
    ar = jnp.arange(pad, dtype=jnp.int32)
    row_pad = jnp.concatenate([edge_index[0], ar % n])
    col_pad = jnp.concatenate([edge_index[1], n + (ar % (_NPAD - n))])
    row2 = jnp.stack([row_pad, row_pad + jnp.int32(n)])
    zeros1 = jnp.zeros((_NPAD // _NS,), _f32)
    zeros128 = jnp.zeros((_K, 128), _f32)
    wcat = jnp.concatenate([W1, Wres], axis=1)
    b1r = b1.reshape(1, 64)
    b2r = b2.reshape(1, 256)
    b3r = b3.reshape(1, 512)
    b4r = b4.reshape(1, 64)
    bresr = bres.reshape(1, 64)
    lngr = ln_g.reshape(1, 64)
    lnbr = ln_b.reshape(1, 64)
    fcngr = fcn_g.reshape(1, 64)
    fcnbr = fcn_b.reshape(1, 64)
    fc1br = fc1_b.reshape(1, 64)
    fc2br = fc2_b.reshape(1, 16)
    batch2 = batch.reshape(n, 1)

    row2d = row_pad.reshape(_EPAD // _K, _K)
    col2d = col_pad.reshape(_EPAD // _K, _K)
    row23d = row2.reshape(2, _EPAD // _K, _K)

    degp = _make_deg_kernel()(col2d, zeros1)
    dcol = (degp[0, :n] + degp[1, :n]).reshape(n, 1)

    dspec = _row_spec(1)
    agg64 = _make_agg_e()
    p0s = _plane_spec(0, 128)
    p1s = _plane_spec(1, 128)
    f0s = p0s
    f1s = p1s

    gs1, res = pl.pallas_call(
        _mm1_body,
        grid=grid,
        in_specs=[_row_spec(128), _full_spec(128, 128), _full_spec(1, 64),
                  dspec],
        out_specs=[_row_spec(128), _row_spec(64)],
        out_shape=[_SDS((n, 128), _f32), _SDS((n, 64), _f32)],
    )(x, wcat, bresr, dcol)

    acc1 = agg64(gs1, row2d, col2d, zeros128)

    gs2 = pl.pallas_call(
        _ew2_body,
        grid=grid,
        in_specs=[_row_spec(128), p0s, p1s, _full_spec(1, 64), dspec],
        out_specs=_row_spec(128),
        out_shape=_SDS((n, 128), _f32),
    )(gs1, acc1, acc1, b1r, dcol)

    acc2 = agg64(gs2, row2d, col2d, zeros128)

    gs3 = pl.pallas_call(
        _mm2_body,
        grid=grid,
        in_specs=[_row_spec(128), p0s, p1s, _full_spec(64, 256),
                  _full_spec(1, 256), dspec],
        out_specs=pl.BlockSpec((2, _BN, 128), lambda i: (0, i, 0)),
        out_shape=_SDS((2, n, 128), _f32),
    )(gs2, acc2, acc2, W2, b2r, dcol)

    gs3cat = gs3.reshape(2 * n, 128)
    acc3 = _make_agg_f()(gs3cat, row23d, col2d, zeros128)

    gs4 = pl.pallas_call(
        _mm34_body,
        grid=grid,
        in_specs=[pl.BlockSpec((2, _BN, 128), lambda i: (0, i, 0)),
                  f0s, f1s, _full_spec(256, 512), _full_spec(1, 512),
                  _full_spec(512, 64), dspec],
        out_specs=_row_spec(128),
        out_shape=_SDS((n, 128), _f32),
    )(gs3, acc3, acc3, W3, b3r, W4, dcol)

    acc4 = agg64(gs4, row2d, col2d, zeros128)

    out = pl.pallas_call(
        _fin_pool_head_body,
        grid=grid,
        in_specs=[_row_spec(128), p0s, p1s, _full_spec(1, 64), _row_spec(64),
                  _full_spec(1, 64), _full_spec(1, 64), dspec,
                  pl.BlockSpec((_BN, 1), lambda i: (i, 0)),
                  _full_spec(64, 64), _full_spec(1, 64), _full_spec(64, 16),
                  _full_spec(1, 16), _full_spec(1, 64), _full_spec(1, 64)],
        out_specs=_full_spec(_GRAPHS, 16),
        out_shape=_SDS((_GRAPHS, 16), _f32),
        scratch_shapes=[pltpu.VMEM((_GRAPHS, 64), _f32)],
        compiler_params=pltpu.CompilerParams(
            dimension_semantics=("arbitrary",)),
    )(gs4, acc4, acc4, b4r, res, lngr, lnbr, dcol, batch2,
      fc1_W, fc1br, fc2_W, fc2br, fcngr, fcnbr)

    return out
